# baseline (speedup 1.0000x reference)
import functools

import jax
import jax.numpy as jnp
from jax import lax
from jax.experimental import pallas as pl
from jax.experimental.pallas import tpu as pltpu
from jax.experimental.pallas import tpu_sc as plsc

N = 10000
D = 128
E = 320000
NLAYERS = 26
NC, NS = 2, 16
CHUNK = 50
NCHUNK = E // NS // CHUNK
BCH = 40
NBLK = NCHUNK // BCH
NBUF = 5
ROWS_PT = 624
ROWS_LAST = N - 15 * ROWS_PT
BM = 400


def _make_agg(width):
  mesh = plsc.VectorSubcoreMesh(
      core_axis_name="c", subcore_axis_name="s", num_cores=NC, num_subcores=NS)

  @functools.partial(
      pl.kernel,
      out_type=(jax.ShapeDtypeStruct((N, width), jnp.float32),
                jax.ShapeDtypeStruct((N, width), jnp.float32)),
      mesh=mesh,
      scratch_types=(
          pltpu.VMEM_SHARED((N, width), jnp.float32),
          pltpu.VMEM((BCH, CHUNK), jnp.int32),
          pltpu.VMEM((BCH, CHUNK), jnp.int32),
          pltpu.VMEM((CHUNK, width), jnp.float32),
          pltpu.VMEM((CHUNK, width), jnp.float32),
          pltpu.VMEM((CHUNK, width), jnp.float32),
          pltpu.VMEM((CHUNK, width), jnp.float32),
          pltpu.VMEM((CHUNK, width), jnp.float32),
          pltpu.SemaphoreType.DMA,
          pltpu.SemaphoreType.DMA,
          pltpu.SemaphoreType.DMA,
          pltpu.SemaphoreType.DMA,
          pltpu.SemaphoreType.DMA,
      ),
  )
  def agg(x1, src1, dst1, x2, src2, dst2, zeros, out1, out2,
          acc, src_idx, dst_idx, r0, r1, r2, r3, r4, g0, g1, g2, g3, g4):
    rows = (r0, r1, r2, r3, r4)
    gsem = (g0, g1, g2, g3, g4)
    c = lax.axis_index("c")
    s = lax.axis_index("s")
    cbase = pl.multiple_of(s * NCHUNK, 8)
    rbase = pl.multiple_of(s * ROWS_PT, 8)

    def rows_fanout(fn):
      @pl.when(s < NS - 1)
      def _():
        fn(pl.ds(rbase, ROWS_PT))

      @pl.when(s == NS - 1)
      def _():
        fn(pl.ds((NS - 1) * ROWS_PT, ROWS_LAST))

    def run(x, src_r, dst_r, out):
      rows_fanout(lambda sl: pltpu.sync_copy(zeros.at[sl], acc.at[sl]))
      plsc.subcore_barrier()

      def block(blk, carry):
        off = pl.multiple_of(cbase + blk * BCH, 8)
        pltpu.sync_copy(src_r.at[pl.ds(off, BCH)], src_idx)
        pltpu.sync_copy(dst_r.at[pl.ds(off, BCH)], dst_idx)

        for b in range(NBUF):
          pltpu.async_copy(x.at[src_idx.at[b]], rows[b], gsem[b])

        def body(g, carry):
          for b in range(NBUF):
            i = NBUF * g + b
            pltpu.make_async_copy(x.at[src_idx.at[i]], rows[b],
                                  gsem[b]).wait()
            pltpu.sync_copy(rows[b], acc.at[dst_idx.at[i]], add=True)

            @pl.when(g < BCH // NBUF - 1)
            def _():
              pltpu.async_copy(x.at[src_idx.at[i + NBUF]], rows[b], gsem[b])
          return carry

        lax.fori_loop(0, BCH // NBUF, body, 0)
        return carry

      lax.fori_loop(0, NBLK, block, 0)
      plsc.subcore_barrier()
      rows_fanout(lambda sl: pltpu.sync_copy(acc.at[sl], out.at[sl]))

    @pl.when(c == 0)
    def _():
      run(x1, src1, dst1, out1)

    @pl.when(c == 1)
    def _():
      run(x2, src2, dst2, out2)

  return agg


_agg_cache = {}


def _get_agg(width):
  if width not in _agg_cache:
    _agg_cache[width] = _make_agg(width)
  return _agg_cache[width]


def _prologue_tc(cnt1, inp1, cnt2, inp2):
  def body(c1, x1, c2, x2, d1_o, p1_o, d2_o, p2_o):
    d1 = lax.rsqrt(c1[:, :1] + 1.0)
    d2 = lax.rsqrt(c2[:, :1] + 1.0)
    d1_o[...] = d1
    d2_o[...] = d2
    p1_o[...] = x1[...] * d1
    p2_o[...] = x2[...] * d2

  grid = (N // BM,)
  row = pl.BlockSpec((BM, D), lambda i: (i, 0))
  col = pl.BlockSpec((BM, 1), lambda i: (i, 0))
  return pl.pallas_call(
      body,
      grid=grid,
      in_specs=[row, row, row, row],
      out_specs=(col, row, col, row),
      out_shape=(jax.ShapeDtypeStruct((N, 1), jnp.float32),
                 jax.ShapeDtypeStruct((N, D), jnp.float32),
                 jax.ShapeDtypeStruct((N, 1), jnp.float32),
                 jax.ShapeDtypeStruct((N, D), jnp.float32)),
  )(cnt1, inp1, cnt2, inp2)


def _layer_tc(agg1, xp1, dis1, w1, b1, agg2, xp2, dis2, w2, b2, last):
  def body(a1, x1, d1, wr1, br1, a2, x2, d2, wr2, br2, o1, o2):
    z1 = (a1[...] + x1[...]) * d1[...]
    z2 = (a2[...] + x2[...]) * d2[...]
    h1 = jnp.dot(z1, wr1[...], preferred_element_type=jnp.float32) + br1[...]
    h2 = jnp.dot(z2, wr2[...], preferred_element_type=jnp.float32) + br2[...]
    if last:
      o1[...] = h1
      o2[...] = h2
    else:
      o1[...] = jnp.maximum(h1, 0.0) * d1[...]
      o2[...] = jnp.maximum(h2, 0.0) * d2[...]

  grid = (N // BM,)
  row = pl.BlockSpec((BM, D), lambda i: (i, 0))
  col = pl.BlockSpec((BM, 1), lambda i: (i, 0))
  wsp = pl.BlockSpec((D, D), lambda i: (0, 0))
  bsp = pl.BlockSpec((1, D), lambda i: (0, 0))
  return pl.pallas_call(
      body,
      grid=grid,
      in_specs=[row, row, col, wsp, bsp, row, row, col, wsp, bsp],
      out_specs=(row, row),
      out_shape=(jax.ShapeDtypeStruct((N, D), jnp.float32),
                 jax.ShapeDtypeStruct((N, D), jnp.float32)),
  )(agg1, xp1, dis1, w1, b1, agg2, xp2, dis2, w2, b2)


def kernel(inp_1, edge_index_1, inp_2, edge_index_2, W1, b1, W2, b2):
  src1 = edge_index_1[0].reshape(E // CHUNK, CHUNK)
  dst1 = edge_index_1[1].reshape(E // CHUNK, CHUNK)
  src2 = edge_index_2[0].reshape(E // CHUNK, CHUNK)
  dst2 = edge_index_2[1].reshape(E // CHUNK, CHUNK)

  zeros_d = jnp.zeros((N, D), jnp.float32)
  ones_d = jnp.ones((N, D), jnp.float32)

  zsrc = jnp.zeros((E // CHUNK, CHUNK), jnp.int32)
  cnt1, cnt2 = _get_agg(D)(ones_d, zsrc, dst1, ones_d, zsrc, dst2, zeros_d)
  dis1, xp1, dis2, xp2 = _prologue_tc(cnt1, inp_1, cnt2, inp_2)

  for j in range(NLAYERS):
    agg1, agg2 = _get_agg(D)(xp1, src1, dst1, xp2, src2, dst2, zeros_d)
    xp1, xp2 = _layer_tc(
        agg1, xp1, dis1, W1[j], b1[j].reshape(1, D),
        agg2, xp2, dis2, W2[j], b2[j].reshape(1, D),
        last=(j == NLAYERS - 1))

  return (xp1, xp2)


# speedup vs baseline: 5.1206x; 5.1206x over previous
import functools

import jax
import jax.numpy as jnp
from jax import lax
from jax.experimental import pallas as pl
from jax.experimental.pallas import tpu as pltpu
from jax.experimental.pallas import tpu_sc as plsc

N = 10000
D = 128
E = 320000
NLAYERS = 26
NC, NS = 2, 16
CHUNK = 50
NCHUNK = E // NS // CHUNK
BCH = 40
NBLK = NCHUNK // BCH
NBUF = 5
ROWS_PT = 624
ROWS_LAST = N - 15 * ROWS_PT
BM = 400


def _make_agg(width):
  mesh = plsc.VectorSubcoreMesh(
      core_axis_name="c", subcore_axis_name="s", num_cores=NC, num_subcores=NS)

  @functools.partial(
      pl.kernel,
      out_type=(jax.ShapeDtypeStruct((N, width), jnp.float32),
                jax.ShapeDtypeStruct((N, width), jnp.float32)),
      mesh=mesh,
      scratch_types=(
          pltpu.VMEM_SHARED((N, width), jnp.float32),
          pltpu.VMEM((BCH, CHUNK), jnp.int32),
          pltpu.VMEM((BCH, CHUNK), jnp.int32),
          pltpu.VMEM((CHUNK, width), jnp.float32),
          pltpu.VMEM((CHUNK, width), jnp.float32),
          pltpu.VMEM((CHUNK, width), jnp.float32),
          pltpu.VMEM((CHUNK, width), jnp.float32),
          pltpu.VMEM((CHUNK, width), jnp.float32),
          pltpu.SemaphoreType.DMA,
          pltpu.SemaphoreType.DMA,
          pltpu.SemaphoreType.DMA,
          pltpu.SemaphoreType.DMA,
          pltpu.SemaphoreType.DMA,
      ),
  )
  def agg(x1, src1, dst1, x2, src2, dst2, zeros, out1, out2,
          acc, src_idx, dst_idx, r0, r1, r2, r3, r4, g0, g1, g2, g3, g4):
    rows = (r0, r1, r2, r3, r4)
    gsem = (g0, g1, g2, g3, g4)
    c = lax.axis_index("c")
    s = lax.axis_index("s")
    cbase = pl.multiple_of(s * NCHUNK, 8)
    rbase = pl.multiple_of(s * ROWS_PT, 8)

    def rows_fanout(fn):
      @pl.when(s < NS - 1)
      def _():
        fn(pl.ds(rbase, ROWS_PT))

      @pl.when(s == NS - 1)
      def _():
        fn(pl.ds((NS - 1) * ROWS_PT, ROWS_LAST))

    def run(x, src_r, dst_r, out):
      rows_fanout(lambda sl: pltpu.sync_copy(zeros.at[sl], acc.at[sl]))
      plsc.subcore_barrier()

      def block(blk, carry):
        off = pl.multiple_of(cbase + blk * BCH, 8)
        pltpu.sync_copy(src_r.at[pl.ds(off, BCH)], src_idx)
        pltpu.sync_copy(dst_r.at[pl.ds(off, BCH)], dst_idx)

        for b in range(NBUF):
          pltpu.async_copy(x.at[src_idx.at[b]], rows[b], gsem[b])

        def body(g, carry):
          for b in range(NBUF):
            i = NBUF * g + b
            pltpu.make_async_copy(x.at[src_idx.at[i]], rows[b],
                                  gsem[b]).wait()
            pltpu.sync_copy(rows[b], acc.at[dst_idx.at[i]], add=True)

            @pl.when(g < BCH // NBUF - 1)
            def _():
              pltpu.async_copy(x.at[src_idx.at[i + NBUF]], rows[b], gsem[b])
          return carry

        lax.fori_loop(0, BCH // NBUF, body, 0)
        return carry

      lax.fori_loop(0, NBLK, block, 0)
      plsc.subcore_barrier()
      rows_fanout(lambda sl: pltpu.sync_copy(acc.at[sl], out.at[sl]))

    @pl.when(c == 0)
    def _():
      run(x1, src1, dst1, out1)

    @pl.when(c == 1)
    def _():
      run(x2, src2, dst2, out2)

  return agg


def _make_deg():
  mesh = plsc.VectorSubcoreMesh(
      core_axis_name="c", subcore_axis_name="s", num_cores=NC, num_subcores=NS)
  NP = 10240
  NW = NP // 16
  HOFF = 640
  HLAST = N - 15 * HOFF

  @functools.partial(
      pl.kernel,
      out_type=(jax.ShapeDtypeStruct((N,), jnp.float32),
                jax.ShapeDtypeStruct((N,), jnp.float32)),
      mesh=mesh,
      compiler_params=pltpu.CompilerParams(needs_layout_passes=False),
      scratch_types=(
          pltpu.VMEM_SHARED((NS, NP), jnp.float32),
          pltpu.VMEM((NP,), jnp.float32),
          pltpu.VMEM((BCH, CHUNK), jnp.int32),
          pltpu.VMEM((HOFF,), jnp.float32),
          pltpu.VMEM((HOFF,), jnp.float32),
      ),
  )
  def deg(dst1, dst2, out1, out2, csh, hist, dst_idx, accb, tmpb):
    c = lax.axis_index("c")
    s = lax.axis_index("s")
    cbase = pl.multiple_of(s * NCHUNK, 8)
    zero16 = jnp.zeros((16,), jnp.float32)
    one16 = jnp.ones((16,), jnp.float32)
    lane = lax.iota(jnp.int32, 16)
    fullmask = lane >= 0
    tailmask = lane >= (16 - (CHUNK - 16 * (CHUNK // 16)))

    def run(dst_r, out):
      def zb(i, carry):
        hist[pl.ds(i * 16, 16)] = zero16
        return carry

      lax.fori_loop(0, NW, zb, 0)

      def block(blk, carry):
        off = pl.multiple_of(cbase + blk * BCH, 8)
        pltpu.sync_copy(dst_r.at[pl.ds(off, BCH)], dst_idx)

        def chunk(i, carry):
          for w in range(CHUNK // 16):
            idxv = dst_idx[i, pl.ds(w * 16, 16)]
            plsc.addupdate_scatter(hist, [idxv], one16, mask=fullmask)
          if CHUNK % 16:
            idxv = dst_idx[i, pl.ds(CHUNK - 16, 16)]
            plsc.addupdate_scatter(hist, [idxv], one16, mask=tailmask)
          return carry

        lax.fori_loop(0, BCH, chunk, 0)
        return carry

      lax.fori_loop(0, NBLK, block, 0)
      pltpu.sync_copy(hist, csh.at[s])
      plsc.subcore_barrier()

      def reduce_range(off, osz):
        def zb2(i, carry):
          accb[pl.ds(i * 16, 16)] = zero16
          return carry

        lax.fori_loop(0, HOFF // 16, zb2, 0)
        for tt in range(NS):
          pltpu.sync_copy(csh.at[tt, pl.ds(off, HOFF)], tmpb)

          def ad(i, carry):
            accb[pl.ds(i * 16, 16)] += tmpb[pl.ds(i * 16, 16)]
            return carry

          lax.fori_loop(0, HOFF // 16, ad, 0)
        pltpu.sync_copy(accb.at[pl.ds(0, osz)], out.at[pl.ds(off, osz)])

      @pl.when(s < NS - 1)
      def _():
        reduce_range(pl.multiple_of(s * HOFF, 128), HOFF)

      @pl.when(s == NS - 1)
      def _():
        reduce_range((NS - 1) * HOFF, HLAST)

    @pl.when(c == 0)
    def _():
      run(dst1, out1)

    @pl.when(c == 1)
    def _():
      run(dst2, out2)

  return deg


_deg_cache = []


def _get_deg():
  if not _deg_cache:
    _deg_cache.append(_make_deg())
  return _deg_cache[0]


_agg_cache = {}


def _get_agg(width):
  if width not in _agg_cache:
    _agg_cache[width] = _make_agg(width)
  return _agg_cache[width]


def _prologue_tc(cnt1, inp1, cnt2, inp2):
  def body(c1, x1, c2, x2, d1_o, p1_o, d2_o, p2_o):
    d1 = lax.rsqrt(c1[...] + 1.0)
    d2 = lax.rsqrt(c2[...] + 1.0)
    d1_o[...] = d1
    d2_o[...] = d2
    p1_o[...] = x1[...] * d1
    p2_o[...] = x2[...] * d2

  grid = (N // BM,)
  row = pl.BlockSpec((BM, D), lambda i: (i, 0))
  col = pl.BlockSpec((BM, 1), lambda i: (i, 0))
  return pl.pallas_call(
      body,
      grid=grid,
      in_specs=[col, row, col, row],
      out_specs=(col, row, col, row),
      out_shape=(jax.ShapeDtypeStruct((N, 1), jnp.float32),
                 jax.ShapeDtypeStruct((N, D), jnp.float32),
                 jax.ShapeDtypeStruct((N, 1), jnp.float32),
                 jax.ShapeDtypeStruct((N, D), jnp.float32)),
  )(cnt1, inp1, cnt2, inp2)


def _layer_tc(agg1, xp1, dis1, w1, b1, agg2, xp2, dis2, w2, b2, last):
  def body(a1, x1, d1, wr1, br1, a2, x2, d2, wr2, br2, o1, o2):
    z1 = (a1[...] + x1[...]) * d1[...]
    z2 = (a2[...] + x2[...]) * d2[...]
    h1 = jnp.dot(z1, wr1[...], preferred_element_type=jnp.float32) + br1[...]
    h2 = jnp.dot(z2, wr2[...], preferred_element_type=jnp.float32) + br2[...]
    if last:
      o1[...] = h1
      o2[...] = h2
    else:
      o1[...] = jnp.maximum(h1, 0.0) * d1[...]
      o2[...] = jnp.maximum(h2, 0.0) * d2[...]

  grid = (N // BM,)
  row = pl.BlockSpec((BM, D), lambda i: (i, 0))
  col = pl.BlockSpec((BM, 1), lambda i: (i, 0))
  wsp = pl.BlockSpec((D, D), lambda i: (0, 0))
  bsp = pl.BlockSpec((1, D), lambda i: (0, 0))
  return pl.pallas_call(
      body,
      grid=grid,
      in_specs=[row, row, col, wsp, bsp, row, row, col, wsp, bsp],
      out_specs=(row, row),
      out_shape=(jax.ShapeDtypeStruct((N, D), jnp.float32),
                 jax.ShapeDtypeStruct((N, D), jnp.float32)),
  )(agg1, xp1, dis1, w1, b1, agg2, xp2, dis2, w2, b2)


def kernel(inp_1, edge_index_1, inp_2, edge_index_2, W1, b1, W2, b2):
  src1 = edge_index_1[0].reshape(E // CHUNK, CHUNK)
  dst1 = edge_index_1[1].reshape(E // CHUNK, CHUNK)
  src2 = edge_index_2[0].reshape(E // CHUNK, CHUNK)
  dst2 = edge_index_2[1].reshape(E // CHUNK, CHUNK)

  zeros_d = jnp.zeros((N, D), jnp.float32)

  cnt1, cnt2 = _get_deg()(dst1, dst2)
  dis1, xp1, dis2, xp2 = _prologue_tc(
      cnt1.reshape(N, 1), inp_1, cnt2.reshape(N, 1), inp_2)

  for j in range(NLAYERS):
    agg1, agg2 = _get_agg(D)(xp1, src1, dst1, xp2, src2, dst2, zeros_d)
    xp1, xp2 = _layer_tc(
        agg1, xp1, dis1, W1[j], b1[j].reshape(1, D),
        agg2, xp2, dis2, W2[j], b2[j].reshape(1, D),
        last=(j == NLAYERS - 1))

  return (xp1, xp2)
